# initial kernel scaffold (unmeasured)
import jax
import jax.numpy as jnp
from jax import lax
from jax.experimental import pallas as pl
from jax.experimental.pallas import tpu as pltpu


def kernel(
    t,
):
    def body(*refs):
        pass

    out_shape = jax.ShapeDtypeStruct(..., jnp.float32)
    return pl.pallas_call(body, out_shape=out_shape)(...)


# baseline (device time: 19362 ns/iter reference)
import jax
import jax.numpy as jnp
from jax import lax
from jax.experimental import pallas as pl
from jax.experimental.pallas import tpu as pltpu

N_DEV = 4


def kernel(t):
    m, n = t.shape

    def body(x_ref, out_ref, acc_ref, comm_ref, send_sems, recv_sems):
        my_pos = lax.axis_index("i")
        p1 = my_pos ^ 1
        p2 = 3 - my_pos

        barrier_sem = pltpu.get_barrier_semaphore()
        for nbr in [p1, p2]:
            pl.semaphore_signal(
                barrier_sem, inc=1,
                device_id=(nbr,), device_id_type=pl.DeviceIdType.MESH,
            )
        pl.semaphore_wait(barrier_sem, 2)

        rdma1 = pltpu.make_async_remote_copy(
            src_ref=x_ref,
            dst_ref=comm_ref.at[0],
            send_sem=send_sems.at[0],
            recv_sem=recv_sems.at[0],
            device_id=(p1,),
            device_id_type=pl.DeviceIdType.MESH,
        )
        rdma1.start()
        rdma1.wait()
        acc_ref[:, :] = x_ref[:, :] + comm_ref[0, :, :]

        rdma2 = pltpu.make_async_remote_copy(
            src_ref=acc_ref,
            dst_ref=comm_ref.at[1],
            send_sem=send_sems.at[1],
            recv_sem=recv_sems.at[1],
            device_id=(p2,),
            device_id_type=pl.DeviceIdType.MESH,
        )
        rdma2.start()
        rdma2.wait()

        s = acc_ref[:, :] + comm_ref[1, :, :]
        r = jnp.maximum(s, 0.0)
        out_ref[:, :] = jnp.tanh(s) * s * s + r * r * r

    return pl.pallas_call(
        body,
        out_shape=jax.ShapeDtypeStruct((m, n), jnp.float32),
        in_specs=[pl.BlockSpec(memory_space=pltpu.VMEM)],
        out_specs=pl.BlockSpec(memory_space=pltpu.VMEM),
        scratch_shapes=[
            pltpu.VMEM((m, n), jnp.float32),
            pltpu.VMEM((2, m, n), jnp.float32),
            pltpu.SemaphoreType.DMA((2,)),
            pltpu.SemaphoreType.DMA((2,)),
        ],
        compiler_params=pltpu.CompilerParams(collective_id=0),
    )(t)


# device time: 15158 ns/iter; 1.2773x vs baseline; 1.2773x over previous
import jax
import jax.numpy as jnp
from jax import lax
from jax.experimental import pallas as pl
from jax.experimental.pallas import tpu as pltpu

N_DEV = 4
CHUNKS = 4


def kernel(t):
    m, n = t.shape
    mc = m // CHUNKS

    def body(x_ref, out_ref, acc_ref, comm1_ref, comm2_ref,
             send1_sems, recv1_sems, send2_sems, recv2_sems):
        my_pos = lax.axis_index("i")
        p1 = my_pos ^ 1
        p2 = 3 - my_pos

        barrier_sem = pltpu.get_barrier_semaphore()
        for nbr in [p1, p2]:
            pl.semaphore_signal(
                barrier_sem, inc=1,
                device_id=(nbr,), device_id_type=pl.DeviceIdType.MESH,
            )
        pl.semaphore_wait(barrier_sem, 2)

        def s1_rdma(c):
            return pltpu.make_async_remote_copy(
                src_ref=x_ref.at[pl.ds(c * mc, mc), :],
                dst_ref=comm1_ref.at[pl.ds(c * mc, mc), :],
                send_sem=send1_sems.at[c],
                recv_sem=recv1_sems.at[c],
                device_id=(p1,),
                device_id_type=pl.DeviceIdType.MESH,
            )

        def s2_rdma(c):
            return pltpu.make_async_remote_copy(
                src_ref=acc_ref.at[pl.ds(c * mc, mc), :],
                dst_ref=comm2_ref.at[pl.ds(c * mc, mc), :],
                send_sem=send2_sems.at[c],
                recv_sem=recv2_sems.at[c],
                device_id=(p2,),
                device_id_type=pl.DeviceIdType.MESH,
            )

        for c in range(CHUNKS):
            s1_rdma(c).start()

        for c in range(CHUNKS):
            s1_rdma(c).wait_recv()
            sl = pl.ds(c * mc, mc)
            acc_ref[sl, :] = x_ref[sl, :] + comm1_ref[sl, :]
            s2_rdma(c).start()

        for c in range(CHUNKS):
            s2_rdma(c).wait_recv()
            sl = pl.ds(c * mc, mc)
            s = acc_ref[sl, :] + comm2_ref[sl, :]
            r = jnp.maximum(s, 0.0)
            out_ref[sl, :] = jnp.tanh(s) * s * s + r * r * r

        for c in range(CHUNKS):
            s1_rdma(c).wait_send()
            s2_rdma(c).wait_send()

    return pl.pallas_call(
        body,
        out_shape=jax.ShapeDtypeStruct((m, n), jnp.float32),
        in_specs=[pl.BlockSpec(memory_space=pltpu.VMEM)],
        out_specs=pl.BlockSpec(memory_space=pltpu.VMEM),
        scratch_shapes=[
            pltpu.VMEM((m, n), jnp.float32),
            pltpu.VMEM((m, n), jnp.float32),
            pltpu.VMEM((m, n), jnp.float32),
            pltpu.SemaphoreType.DMA((CHUNKS,)),
            pltpu.SemaphoreType.DMA((CHUNKS,)),
            pltpu.SemaphoreType.DMA((CHUNKS,)),
            pltpu.SemaphoreType.DMA((CHUNKS,)),
        ],
        compiler_params=pltpu.CompilerParams(collective_id=0),
    )(t)


# device time: 14561 ns/iter; 1.3297x vs baseline; 1.0410x over previous
import jax
import jax.numpy as jnp
from jax import lax
from jax.experimental import pallas as pl
from jax.experimental.pallas import tpu as pltpu

N_DEV = 4
CHUNKS = 8


def kernel(t):
    m, n = t.shape
    mc = m // CHUNKS

    def body(x_ref, out_ref, acc_ref, comm1_ref, comm2_ref,
             send1_sems, recv1_sems, send2_sems, recv2_sems):
        my_pos = lax.axis_index("i")
        p1 = my_pos ^ 1
        p2 = 3 - my_pos

        barrier_sem = pltpu.get_barrier_semaphore()
        for nbr in [p1, p2]:
            pl.semaphore_signal(
                barrier_sem, inc=1,
                device_id=(nbr,), device_id_type=pl.DeviceIdType.MESH,
            )
        pl.semaphore_wait(barrier_sem, 2)

        def s1_rdma(c):
            return pltpu.make_async_remote_copy(
                src_ref=x_ref.at[pl.ds(c * mc, mc), :],
                dst_ref=comm1_ref.at[pl.ds(c * mc, mc), :],
                send_sem=send1_sems.at[c],
                recv_sem=recv1_sems.at[c],
                device_id=(p1,),
                device_id_type=pl.DeviceIdType.MESH,
            )

        def s2_rdma(c):
            return pltpu.make_async_remote_copy(
                src_ref=acc_ref.at[pl.ds(c * mc, mc), :],
                dst_ref=comm2_ref.at[pl.ds(c * mc, mc), :],
                send_sem=send2_sems.at[c],
                recv_sem=recv2_sems.at[c],
                device_id=(p2,),
                device_id_type=pl.DeviceIdType.MESH,
            )

        for c in range(CHUNKS):
            s1_rdma(c).start()

        for c in range(CHUNKS):
            s1_rdma(c).wait_recv()
            sl = pl.ds(c * mc, mc)
            acc_ref[sl, :] = x_ref[sl, :] + comm1_ref[sl, :]
            s2_rdma(c).start()

        for c in range(CHUNKS):
            s2_rdma(c).wait_recv()
            sl = pl.ds(c * mc, mc)
            s = acc_ref[sl, :] + comm2_ref[sl, :]
            r = jnp.maximum(s, 0.0)
            out_ref[sl, :] = jnp.tanh(s) * s * s + r * r * r

        for c in range(CHUNKS):
            s1_rdma(c).wait_send()
            s2_rdma(c).wait_send()

    return pl.pallas_call(
        body,
        out_shape=jax.ShapeDtypeStruct((m, n), jnp.float32),
        in_specs=[pl.BlockSpec(memory_space=pltpu.VMEM)],
        out_specs=pl.BlockSpec(memory_space=pltpu.VMEM),
        scratch_shapes=[
            pltpu.VMEM((m, n), jnp.float32),
            pltpu.VMEM((m, n), jnp.float32),
            pltpu.VMEM((m, n), jnp.float32),
            pltpu.SemaphoreType.DMA((CHUNKS,)),
            pltpu.SemaphoreType.DMA((CHUNKS,)),
            pltpu.SemaphoreType.DMA((CHUNKS,)),
            pltpu.SemaphoreType.DMA((CHUNKS,)),
        ],
        compiler_params=pltpu.CompilerParams(collective_id=0),
    )(t)


# device time: 12625 ns/iter; 1.5336x vs baseline; 1.1533x over previous
import jax
import jax.numpy as jnp
from jax import lax
from jax.experimental import pallas as pl
from jax.experimental.pallas import tpu as pltpu

N_DEV = 4
CH = 4


def kernel(t):
    m, n = t.shape
    half = m // 2
    hc = half // CH

    def body(x_ref, out_ref, acc_ref, comm1_ref, comm2_ref,
             send1_sems, recv1_sems, send2_sems, recv2_sems):
        my_pos = lax.axis_index("i")
        p1 = my_pos ^ 1
        p2 = 3 - my_pos

        barrier_sem = pltpu.get_barrier_semaphore()
        for nbr in [p1, p2]:
            pl.semaphore_signal(
                barrier_sem, inc=1,
                device_id=(nbr,), device_id_type=pl.DeviceIdType.MESH,
            )
        pl.semaphore_wait(barrier_sem, 2)

        def row0(stream, c):
            return stream * half + c * hc

        def s1_rdma(stream, c):
            sl = pl.ds(row0(stream, c), hc)
            return pltpu.make_async_remote_copy(
                src_ref=x_ref.at[sl, :],
                dst_ref=comm1_ref.at[sl, :],
                send_sem=send1_sems.at[stream, c],
                recv_sem=recv1_sems.at[stream, c],
                device_id=(p1 if stream == 0 else p2,),
                device_id_type=pl.DeviceIdType.MESH,
            )

        def s2_rdma(stream, c):
            sl = pl.ds(row0(stream, c), hc)
            return pltpu.make_async_remote_copy(
                src_ref=acc_ref.at[sl, :],
                dst_ref=comm2_ref.at[sl, :],
                send_sem=send2_sems.at[stream, c],
                recv_sem=recv2_sems.at[stream, c],
                device_id=(p2 if stream == 0 else p1,),
                device_id_type=pl.DeviceIdType.MESH,
            )

        for c in range(CH):
            s1_rdma(0, c).start()
            s1_rdma(1, c).start()

        for c in range(CH):
            for stream in (0, 1):
                s1_rdma(stream, c).wait_recv()
                sl = pl.ds(row0(stream, c), hc)
                acc_ref[sl, :] = x_ref[sl, :] + comm1_ref[sl, :]
                s2_rdma(stream, c).start()

        for c in range(CH):
            for stream in (0, 1):
                s2_rdma(stream, c).wait_recv()
                sl = pl.ds(row0(stream, c), hc)
                s = acc_ref[sl, :] + comm2_ref[sl, :]
                r = jnp.maximum(s, 0.0)
                out_ref[sl, :] = jnp.tanh(s) * s * s + r * r * r

        for c in range(CH):
            for stream in (0, 1):
                s1_rdma(stream, c).wait_send()
                s2_rdma(stream, c).wait_send()

    return pl.pallas_call(
        body,
        out_shape=jax.ShapeDtypeStruct((m, n), jnp.float32),
        in_specs=[pl.BlockSpec(memory_space=pltpu.VMEM)],
        out_specs=pl.BlockSpec(memory_space=pltpu.VMEM),
        scratch_shapes=[
            pltpu.VMEM((m, n), jnp.float32),
            pltpu.VMEM((m, n), jnp.float32),
            pltpu.VMEM((m, n), jnp.float32),
            pltpu.SemaphoreType.DMA((2, CH)),
            pltpu.SemaphoreType.DMA((2, CH)),
            pltpu.SemaphoreType.DMA((2, CH)),
            pltpu.SemaphoreType.DMA((2, CH)),
        ],
        compiler_params=pltpu.CompilerParams(collective_id=0),
    )(t)


# device time: 12499 ns/iter; 1.5491x vs baseline; 1.0101x over previous
import jax
import jax.numpy as jnp
from jax import lax
from jax.experimental import pallas as pl
from jax.experimental.pallas import tpu as pltpu

N_DEV = 4
CH = 2


def kernel(t):
    m, n = t.shape
    half = m // 2
    hc = half // CH

    def body(x_ref, out_ref, acc_ref, comm1_ref, comm2_ref,
             send1_sems, recv1_sems, send2_sems, recv2_sems):
        my_pos = lax.axis_index("i")
        p1 = my_pos ^ 1
        p2 = 3 - my_pos

        barrier_sem = pltpu.get_barrier_semaphore()
        for nbr in [p1, p2]:
            pl.semaphore_signal(
                barrier_sem, inc=1,
                device_id=(nbr,), device_id_type=pl.DeviceIdType.MESH,
            )
        pl.semaphore_wait(barrier_sem, 2)

        def row0(stream, c):
            return stream * half + c * hc

        def s1_rdma(stream, c):
            sl = pl.ds(row0(stream, c), hc)
            return pltpu.make_async_remote_copy(
                src_ref=x_ref.at[sl, :],
                dst_ref=comm1_ref.at[sl, :],
                send_sem=send1_sems.at[stream, c],
                recv_sem=recv1_sems.at[stream, c],
                device_id=(p1 if stream == 0 else p2,),
                device_id_type=pl.DeviceIdType.MESH,
            )

        def s2_rdma(stream, c):
            sl = pl.ds(row0(stream, c), hc)
            return pltpu.make_async_remote_copy(
                src_ref=acc_ref.at[sl, :],
                dst_ref=comm2_ref.at[sl, :],
                send_sem=send2_sems.at[stream, c],
                recv_sem=recv2_sems.at[stream, c],
                device_id=(p2 if stream == 0 else p1,),
                device_id_type=pl.DeviceIdType.MESH,
            )

        for c in range(CH):
            s1_rdma(0, c).start()
            s1_rdma(1, c).start()

        for c in range(CH):
            for stream in (0, 1):
                s1_rdma(stream, c).wait_recv()
                sl = pl.ds(row0(stream, c), hc)
                acc_ref[sl, :] = x_ref[sl, :] + comm1_ref[sl, :]
                s2_rdma(stream, c).start()

        for c in range(CH):
            for stream in (0, 1):
                s2_rdma(stream, c).wait_recv()
                sl = pl.ds(row0(stream, c), hc)
                s = acc_ref[sl, :] + comm2_ref[sl, :]
                r = jnp.maximum(s, 0.0)
                out_ref[sl, :] = jnp.tanh(s) * s * s + r * r * r

        for c in range(CH):
            for stream in (0, 1):
                s1_rdma(stream, c).wait_send()
                s2_rdma(stream, c).wait_send()

    return pl.pallas_call(
        body,
        out_shape=jax.ShapeDtypeStruct((m, n), jnp.float32),
        in_specs=[pl.BlockSpec(memory_space=pltpu.VMEM)],
        out_specs=pl.BlockSpec(memory_space=pltpu.VMEM),
        scratch_shapes=[
            pltpu.VMEM((m, n), jnp.float32),
            pltpu.VMEM((m, n), jnp.float32),
            pltpu.VMEM((m, n), jnp.float32),
            pltpu.SemaphoreType.DMA((2, CH)),
            pltpu.SemaphoreType.DMA((2, CH)),
            pltpu.SemaphoreType.DMA((2, CH)),
            pltpu.SemaphoreType.DMA((2, CH)),
        ],
        compiler_params=pltpu.CompilerParams(collective_id=0),
    )(t)
